# baseline (speedup 1.0000x reference)
import functools

import jax
import jax.numpy as jnp
from jax import lax
from jax.experimental import pallas as pl
from jax.experimental.pallas import tpu as pltpu
from jax.experimental.pallas import tpu_sc as plsc

_FC = 8
_LB = 2048


def _prep_body(c1T_ref, cen_ref, W1pT_ref, b1p_ref, W1v_ref, b1v_ref,
               ncT_ref, val_ref, *, fc, sc):
    ptT = jnp.dot(W1pT_ref[:], c1T_ref[0],
                  preferred_element_type=jnp.float32) + b1p_ref[:]
    blocks = []
    for f in range(fc):
        blk = ptT[f * sc:(f + 1) * sc, :]
        nrm = jnp.sqrt(jnp.sum(blk * blk, axis=0, keepdims=True))
        blocks.append(blk / jnp.maximum(nrm, 1e-12))
    ncT_ref[0] = jnp.concatenate(blocks, axis=0)
    val_ref[0] = jnp.dot(cen_ref[0], W1v_ref[:],
                         preferred_element_type=jnp.float32) + b1v_ref[:]


def _route_body(x_ref, W0_ref, b0_ref, ncT_ref, ab_ref, mi_ref, mv_ref,
                *, fc, sc, s):
    xp = jnp.dot(x_ref[0], W0_ref[:],
                 preferred_element_type=jnp.float32) + b0_ref[:]
    a = ab_ref[0, 0]
    b = ab_ref[0, 1]
    n_idx = pl.program_id(0)
    iota_f32 = lax.broadcasted_iota(
        jnp.int32, (xp.shape[0], s), 1).astype(jnp.float32)
    mvs, mis = [], []
    for f in range(fc):
        ch = xp[:, f * sc:(f + 1) * sc]
        nrm = jnp.sqrt(jnp.sum(ch * ch, axis=1, keepdims=True))
        nx = ch / jnp.maximum(nrm, 1e-12)
        sims = jnp.dot(nx, ncT_ref[0, f * sc:(f + 1) * sc, :],
                       preferred_element_type=jnp.float32)
        sims = jax.nn.sigmoid(a * sims + b)
        mx = jnp.max(sims, axis=1, keepdims=True)
        am = jnp.min(jnp.where(sims == mx, iota_f32, float(s)),
                     axis=1, keepdims=True)
        am = jnp.minimum(am, float(s - 1))
        mvs.append(mx)
        mis.append(am * float(fc)
                   + (n_idx * (s * fc) + f).astype(jnp.float32))
    mv_ref[0] = jnp.concatenate(mvs, axis=1)
    mi_ref[0] = jnp.concatenate(mis, axis=1).astype(jnp.int32)


def _proj_body(d_ref, mv_ref, Wm_ref, bm_ref, out_ref, *, fc, sc):
    mvb = mv_ref[0]
    parts = [d_ref[f, 0] * mvb[:, f:f + 1] for f in range(fc)]
    sd = jnp.concatenate(parts, axis=1)
    out_ref[0] = jnp.dot(sd, Wm_ref[:],
                         preferred_element_type=jnp.float32) + bm_ref[:]


def kernel(x0, center1, W0, b0, W1, b1, Wm, bm, alpha, beta):
    fc = _FC
    n, l, c = x0.shape
    s = center1.shape[1]
    h = W0.shape[1]
    sc = h // fc

    c1T = jnp.swapaxes(center1, 1, 2)
    W1r = W1.reshape(c, fc, 2 * sc)
    W1pT = W1r[:, :, :sc].transpose(1, 2, 0).reshape(fc * sc, c)
    W1v = W1r[:, :, sc:].reshape(c, fc * sc)
    b1r = b1.reshape(fc, 2 * sc)
    b1p = b1r[:, :sc].reshape(fc * sc, 1)
    b1v = b1r[:, sc:].reshape(1, fc * sc)
    b0r = b0.reshape(1, h)
    bmr = bm.reshape(1, c)
    ab = jnp.concatenate([alpha, beta]).reshape(1, 2)

    ncT, val = pl.pallas_call(
        functools.partial(_prep_body, fc=fc, sc=sc),
        grid=(n,),
        in_specs=[
            pl.BlockSpec((1, c, s), lambda i: (i, 0, 0)),
            pl.BlockSpec((1, s, c), lambda i: (i, 0, 0)),
            pl.BlockSpec((h, c), lambda i: (0, 0)),
            pl.BlockSpec((h, 1), lambda i: (0, 0)),
            pl.BlockSpec((c, h), lambda i: (0, 0)),
            pl.BlockSpec((1, h), lambda i: (0, 0)),
        ],
        out_specs=[
            pl.BlockSpec((1, h, s), lambda i: (i, 0, 0)),
            pl.BlockSpec((1, s, h), lambda i: (i, 0, 0)),
        ],
        out_shape=[
            jax.ShapeDtypeStruct((n, h, s), jnp.float32),
            jax.ShapeDtypeStruct((n, s, h), jnp.float32),
        ],
    )(c1T, center1, W1pT, b1p, W1v, b1v)

    nlb = l // _LB
    mi, mv = pl.pallas_call(
        functools.partial(_route_body, fc=fc, sc=sc, s=s),
        grid=(n, nlb),
        in_specs=[
            pl.BlockSpec((1, _LB, c), lambda i, j: (i, j, 0)),
            pl.BlockSpec((c, h), lambda i, j: (0, 0)),
            pl.BlockSpec((1, h), lambda i, j: (0, 0)),
            pl.BlockSpec((1, h, s), lambda i, j: (i, 0, 0)),
            pl.BlockSpec(memory_space=pltpu.SMEM),
        ],
        out_specs=[
            pl.BlockSpec((1, _LB, fc), lambda i, j: (i, j, 0)),
            pl.BlockSpec((1, _LB, fc), lambda i, j: (i, j, 0)),
        ],
        out_shape=[
            jax.ShapeDtypeStruct((n, l, fc), jnp.int32),
            jax.ShapeDtypeStruct((n, l, fc), jnp.float32),
        ],
    )(x0, W0, b0r, ncT, ab)

    tbl = val.reshape(n * s * fc, sc)
    idx = jnp.transpose(mi, (2, 0, 1)).reshape(n * l * fc)
    B = n * l * fc
    NW = 32
    bpw = B // NW
    CH = 128
    nch = bpw // CH
    mesh = plsc.VectorSubcoreMesh(core_axis_name="c", subcore_axis_name="s")

    @functools.partial(
        pl.kernel, mesh=mesh,
        out_type=jax.ShapeDtypeStruct((B, sc), jnp.float32),
        scratch_types=[
            pltpu.VMEM((2, CH), jnp.int32),
            pltpu.VMEM((2, CH, sc), jnp.float32),
            pltpu.SemaphoreType.DMA,
            pltpu.SemaphoreType.DMA,
            pltpu.SemaphoreType.DMA,
            pltpu.SemaphoreType.DMA,
        ],
    )
    def _gather(tbl_hbm, idx_hbm, out_hbm, idx_v, rows_v, g0, g1, s0, s1):
        wid = lax.axis_index("s") * 2 + lax.axis_index("c")
        base = wid * bpw
        gs = (g0, g1)
        ss = (s0, s1)
        hg = [None, None]
        hs = [None, None]
        pltpu.sync_copy(idx_hbm.at[pl.ds(base, CH)], idx_v.at[0])
        hg[0] = pltpu.async_copy(tbl_hbm.at[idx_v.at[0]], rows_v.at[0], gs[0])
        for i in range(nch):
            bb = i % 2
            nb = (i + 1) % 2
            if i + 1 < nch:
                if hs[nb] is not None:
                    hs[nb].wait()
                pltpu.sync_copy(idx_hbm.at[pl.ds(base + (i + 1) * CH, CH)],
                                idx_v.at[nb])
                hg[nb] = pltpu.async_copy(tbl_hbm.at[idx_v.at[nb]],
                                          rows_v.at[nb], gs[nb])
            hg[bb].wait()
            hs[bb] = pltpu.async_copy(rows_v.at[bb],
                                      out_hbm.at[pl.ds(base + i * CH, CH)],
                                      ss[bb])
        hs[0].wait()
        hs[1].wait()

    disp = _gather(tbl, idx)

    dispr = disp.reshape(fc, n, l, sc)
    out = pl.pallas_call(
        functools.partial(_proj_body, fc=fc, sc=sc),
        grid=(n, nlb),
        in_specs=[
            pl.BlockSpec((fc, 1, _LB, sc), lambda i, j: (0, i, j, 0)),
            pl.BlockSpec((1, _LB, fc), lambda i, j: (i, j, 0)),
            pl.BlockSpec((h, c), lambda i, j: (0, 0)),
            pl.BlockSpec((1, c), lambda i, j: (0, 0)),
        ],
        out_specs=pl.BlockSpec((1, _LB, c), lambda i, j: (i, j, 0)),
        out_shape=jax.ShapeDtypeStruct((n, l, c), jnp.float32),
    )(dispr, mv, Wm, bmr)
    return out


# speedup vs baseline: 1.0040x; 1.0040x over previous
import functools

import jax
import jax.numpy as jnp
from jax import lax
from jax.experimental import pallas as pl
from jax.experimental.pallas import tpu as pltpu
from jax.experimental.pallas import tpu_sc as plsc

_FC = 8
_LB = 1024


def _prep_body(c1T_ref, cen_ref, W1pT_ref, b1p_ref, W1v_ref, b1v_ref,
               ncT_ref, val_ref, *, fc, sc):
    ptT = jnp.dot(W1pT_ref[:], c1T_ref[0],
                  preferred_element_type=jnp.float32) + b1p_ref[:]
    blocks = []
    for f in range(fc):
        blk = ptT[f * sc:(f + 1) * sc, :]
        nrm = jnp.sqrt(jnp.sum(blk * blk, axis=0, keepdims=True))
        blocks.append(blk / jnp.maximum(nrm, 1e-12))
    ncT_ref[0] = jnp.concatenate(blocks, axis=0)
    val_ref[0] = jnp.dot(cen_ref[0], W1v_ref[:],
                         preferred_element_type=jnp.float32) + b1v_ref[:]


def _route_body(x_ref, W0_ref, b0_ref, ncT_ref, ab_ref, mi_ref, mv_ref,
                *, fc, sc, s):
    xp = jnp.dot(x_ref[0], W0_ref[:],
                 preferred_element_type=jnp.float32) + b0_ref[:]
    a = ab_ref[0, 0]
    b = ab_ref[0, 1]
    n_idx = pl.program_id(0)
    iota_f32 = lax.broadcasted_iota(
        jnp.int32, (xp.shape[0], s), 1).astype(jnp.float32)
    mvs, mis = [], []
    for f in range(fc):
        ch = xp[:, f * sc:(f + 1) * sc]
        nrm = jnp.sqrt(jnp.sum(ch * ch, axis=1, keepdims=True))
        nx = ch / jnp.maximum(nrm, 1e-12)
        sims = jnp.dot(nx, ncT_ref[0, f * sc:(f + 1) * sc, :],
                       preferred_element_type=jnp.float32)
        sims = jax.nn.sigmoid(a * sims + b)
        mx = jnp.max(sims, axis=1, keepdims=True)
        am = jnp.min(jnp.where(sims == mx, iota_f32, float(s)),
                     axis=1, keepdims=True)
        am = jnp.minimum(am, float(s - 1))
        mvs.append(mx)
        mis.append(am * float(fc)
                   + (n_idx * (s * fc) + f).astype(jnp.float32))
    mv_ref[0] = jnp.concatenate(mvs, axis=1)
    mi_ref[0] = jnp.concatenate(mis, axis=1).astype(jnp.int32)


def _proj_body(d_ref, mv_ref, Wm_ref, bm_ref, out_ref, *, fc, sc):
    mvb = mv_ref[0]
    parts = [d_ref[f, 0] * mvb[:, f:f + 1] for f in range(fc)]
    sd = jnp.concatenate(parts, axis=1)
    out_ref[0] = jnp.dot(sd, Wm_ref[:],
                         preferred_element_type=jnp.float32) + bm_ref[:]


def kernel(x0, center1, W0, b0, W1, b1, Wm, bm, alpha, beta):
    fc = _FC
    n, l, c = x0.shape
    s = center1.shape[1]
    h = W0.shape[1]
    sc = h // fc

    c1T = jnp.swapaxes(center1, 1, 2)
    W1r = W1.reshape(c, fc, 2 * sc)
    W1pT = W1r[:, :, :sc].transpose(1, 2, 0).reshape(fc * sc, c)
    W1v = W1r[:, :, sc:].reshape(c, fc * sc)
    b1r = b1.reshape(fc, 2 * sc)
    b1p = b1r[:, :sc].reshape(fc * sc, 1)
    b1v = b1r[:, sc:].reshape(1, fc * sc)
    b0r = b0.reshape(1, h)
    bmr = bm.reshape(1, c)
    ab = jnp.concatenate([alpha, beta]).reshape(1, 2)

    ncT, val = pl.pallas_call(
        functools.partial(_prep_body, fc=fc, sc=sc),
        grid=(n,),
        in_specs=[
            pl.BlockSpec((1, c, s), lambda i: (i, 0, 0)),
            pl.BlockSpec((1, s, c), lambda i: (i, 0, 0)),
            pl.BlockSpec((h, c), lambda i: (0, 0)),
            pl.BlockSpec((h, 1), lambda i: (0, 0)),
            pl.BlockSpec((c, h), lambda i: (0, 0)),
            pl.BlockSpec((1, h), lambda i: (0, 0)),
        ],
        out_specs=[
            pl.BlockSpec((1, h, s), lambda i: (i, 0, 0)),
            pl.BlockSpec((1, s, h), lambda i: (i, 0, 0)),
        ],
        out_shape=[
            jax.ShapeDtypeStruct((n, h, s), jnp.float32),
            jax.ShapeDtypeStruct((n, s, h), jnp.float32),
        ],
    )(c1T, center1, W1pT, b1p, W1v, b1v)

    nlb = l // _LB
    mi, mv = pl.pallas_call(
        functools.partial(_route_body, fc=fc, sc=sc, s=s),
        grid=(n, nlb),
        in_specs=[
            pl.BlockSpec((1, _LB, c), lambda i, j: (i, j, 0)),
            pl.BlockSpec((c, h), lambda i, j: (0, 0)),
            pl.BlockSpec((1, h), lambda i, j: (0, 0)),
            pl.BlockSpec((1, h, s), lambda i, j: (i, 0, 0)),
            pl.BlockSpec(memory_space=pltpu.SMEM),
        ],
        out_specs=[
            pl.BlockSpec((1, _LB, fc), lambda i, j: (i, j, 0)),
            pl.BlockSpec((1, _LB, fc), lambda i, j: (i, j, 0)),
        ],
        out_shape=[
            jax.ShapeDtypeStruct((n, l, fc), jnp.int32),
            jax.ShapeDtypeStruct((n, l, fc), jnp.float32),
        ],
    )(x0, W0, b0r, ncT, ab)

    tbl = val.reshape(n * s * fc, sc)
    idx = jnp.transpose(mi, (2, 0, 1)).reshape(n * l * fc)
    B = n * l * fc
    NW = 32
    bpw = B // NW
    CH = 128
    nch = bpw // CH
    mesh = plsc.VectorSubcoreMesh(core_axis_name="c", subcore_axis_name="s")

    @functools.partial(
        pl.kernel, mesh=mesh,
        out_type=jax.ShapeDtypeStruct((B, sc), jnp.float32),
        scratch_types=[
            pltpu.VMEM((2, CH), jnp.int32),
            pltpu.VMEM((2, CH, sc), jnp.float32),
            pltpu.SemaphoreType.DMA,
            pltpu.SemaphoreType.DMA,
            pltpu.SemaphoreType.DMA,
            pltpu.SemaphoreType.DMA,
        ],
    )
    def _gather(tbl_hbm, idx_hbm, out_hbm, idx_v, rows_v, g0, g1, s0, s1):
        wid = lax.axis_index("s") * 2 + lax.axis_index("c")
        base = wid * bpw
        gs = (g0, g1)
        ss = (s0, s1)
        hg = [None, None]
        hs = [None, None]
        pltpu.sync_copy(idx_hbm.at[pl.ds(base, CH)], idx_v.at[0])
        hg[0] = pltpu.async_copy(tbl_hbm.at[idx_v.at[0]], rows_v.at[0], gs[0])
        for i in range(nch):
            bb = i % 2
            nb = (i + 1) % 2
            if i + 1 < nch:
                if hs[nb] is not None:
                    hs[nb].wait()
                pltpu.sync_copy(idx_hbm.at[pl.ds(base + (i + 1) * CH, CH)],
                                idx_v.at[nb])
                hg[nb] = pltpu.async_copy(tbl_hbm.at[idx_v.at[nb]],
                                          rows_v.at[nb], gs[nb])
            hg[bb].wait()
            hs[bb] = pltpu.async_copy(rows_v.at[bb],
                                      out_hbm.at[pl.ds(base + i * CH, CH)],
                                      ss[bb])
        hs[0].wait()
        hs[1].wait()

    disp = _gather(tbl, idx)

    dispr = disp.reshape(fc, n, l, sc)
    out = pl.pallas_call(
        functools.partial(_proj_body, fc=fc, sc=sc),
        grid=(n, nlb),
        in_specs=[
            pl.BlockSpec((fc, 1, _LB, sc), lambda i, j: (0, i, j, 0)),
            pl.BlockSpec((1, _LB, fc), lambda i, j: (i, j, 0)),
            pl.BlockSpec((h, c), lambda i, j: (0, 0)),
            pl.BlockSpec((1, c), lambda i, j: (0, 0)),
        ],
        out_specs=pl.BlockSpec((1, _LB, c), lambda i, j: (i, j, 0)),
        out_shape=jax.ShapeDtypeStruct((n, l, c), jnp.float32),
    )(dispr, mv, Wm, bmr)
    return out


# speedup vs baseline: 1.0972x; 1.0928x over previous
import functools

import jax
import jax.numpy as jnp
from jax import lax
from jax.experimental import pallas as pl
from jax.experimental.pallas import tpu as pltpu
from jax.experimental.pallas import tpu_sc as plsc

_FC = 8
_LB = 1024


def _prep_body(c1T_ref, cen_ref, W1_ref, b1p_ref, b1v_ref,
               ncT_ref, val_ref, *, fc, sc):
    blocks, vals = [], []
    for f in range(fc):
        w1p = W1_ref[:, f * 2 * sc:f * 2 * sc + sc]
        blk = lax.dot_general(w1p, c1T_ref[0], (((0,), (0,)), ((), ())),
                              preferred_element_type=jnp.float32)
        blk = blk + b1p_ref[f * sc:(f + 1) * sc, :]
        nrm = jnp.sqrt(jnp.sum(blk * blk, axis=0, keepdims=True))
        blocks.append(blk / jnp.maximum(nrm, 1e-12))
        w1v = W1_ref[:, f * 2 * sc + sc:(f + 1) * 2 * sc]
        vals.append(jnp.dot(cen_ref[0], w1v,
                            preferred_element_type=jnp.float32))
    ncT_ref[0] = jnp.concatenate(blocks, axis=0)
    val_ref[0] = jnp.concatenate(vals, axis=1) + b1v_ref[:]


def _route_body(x_ref, W0_ref, b0_ref, ncT_ref, ab_ref, mi_ref, mv_ref,
                *, fc, sc, s):
    xp = jnp.dot(x_ref[0], W0_ref[:],
                 preferred_element_type=jnp.float32) + b0_ref[:]
    a = ab_ref[0, 0]
    b = ab_ref[0, 1]
    n_idx = pl.program_id(0)
    iota_f32 = lax.broadcasted_iota(
        jnp.int32, (xp.shape[0], s), 1).astype(jnp.float32)
    mvs, mis = [], []
    for f in range(fc):
        ch = xp[:, f * sc:(f + 1) * sc]
        nrm = jnp.sqrt(jnp.sum(ch * ch, axis=1, keepdims=True))
        nx = ch / jnp.maximum(nrm, 1e-12)
        sims = jnp.dot(nx, ncT_ref[0, f * sc:(f + 1) * sc, :],
                       preferred_element_type=jnp.float32)
        sims = jax.nn.sigmoid(a * sims + b)
        mx = jnp.max(sims, axis=1, keepdims=True)
        am = jnp.min(jnp.where(sims == mx, iota_f32, float(s)),
                     axis=1, keepdims=True)
        am = jnp.minimum(am, float(s - 1))
        mvs.append(mx)
        mis.append(am * float(fc)
                   + (n_idx * (s * fc) + f).astype(jnp.float32))
    mv_ref[0] = jnp.concatenate(mvs, axis=1)
    mi_ref[0] = jnp.concatenate(mis, axis=1).astype(jnp.int32)


def _proj_body(d_ref, mv_ref, Wm_ref, bm_ref, out_ref, *, fc, sc):
    mvb = mv_ref[0]
    parts = [d_ref[f, 0] * mvb[:, f:f + 1] for f in range(fc)]
    sd = jnp.concatenate(parts, axis=1)
    out_ref[0] = jnp.dot(sd, Wm_ref[:],
                         preferred_element_type=jnp.float32) + bm_ref[:]


def kernel(x0, center1, W0, b0, W1, b1, Wm, bm, alpha, beta):
    fc = _FC
    n, l, c = x0.shape
    s = center1.shape[1]
    h = W0.shape[1]
    sc = h // fc

    c1T = jnp.swapaxes(center1, 1, 2)
    b1r = b1.reshape(fc, 2 * sc)
    b1p = b1r[:, :sc].reshape(fc * sc, 1)
    b1v = b1r[:, sc:].reshape(1, fc * sc)
    b0r = b0.reshape(1, h)
    bmr = bm.reshape(1, c)
    ab = jnp.concatenate([alpha, beta]).reshape(1, 2)

    ncT, val = pl.pallas_call(
        functools.partial(_prep_body, fc=fc, sc=sc),
        grid=(n,),
        in_specs=[
            pl.BlockSpec((1, c, s), lambda i: (i, 0, 0)),
            pl.BlockSpec((1, s, c), lambda i: (i, 0, 0)),
            pl.BlockSpec((c, 2 * h), lambda i: (0, 0)),
            pl.BlockSpec((h, 1), lambda i: (0, 0)),
            pl.BlockSpec((1, h), lambda i: (0, 0)),
        ],
        out_specs=[
            pl.BlockSpec((1, h, s), lambda i: (i, 0, 0)),
            pl.BlockSpec((1, s, h), lambda i: (i, 0, 0)),
        ],
        out_shape=[
            jax.ShapeDtypeStruct((n, h, s), jnp.float32),
            jax.ShapeDtypeStruct((n, s, h), jnp.float32),
        ],
    )(c1T, center1, W1, b1p, b1v)

    nlb = l // _LB
    mi, mv = pl.pallas_call(
        functools.partial(_route_body, fc=fc, sc=sc, s=s),
        grid=(n, nlb),
        in_specs=[
            pl.BlockSpec((1, _LB, c), lambda i, j: (i, j, 0)),
            pl.BlockSpec((c, h), lambda i, j: (0, 0)),
            pl.BlockSpec((1, h), lambda i, j: (0, 0)),
            pl.BlockSpec((1, h, s), lambda i, j: (i, 0, 0)),
            pl.BlockSpec(memory_space=pltpu.SMEM),
        ],
        out_specs=[
            pl.BlockSpec((1, _LB, fc), lambda i, j: (i, j, 0)),
            pl.BlockSpec((1, _LB, fc), lambda i, j: (i, j, 0)),
        ],
        out_shape=[
            jax.ShapeDtypeStruct((n, l, fc), jnp.int32),
            jax.ShapeDtypeStruct((n, l, fc), jnp.float32),
        ],
    )(x0, W0, b0r, ncT, ab)

    tbl = val.reshape(n * s * fc, sc)
    idx = jnp.transpose(mi, (2, 0, 1)).reshape(n * l * fc)
    B = n * l * fc
    NW = 32
    bpw = B // NW
    CH = 128
    nch = bpw // CH
    mesh = plsc.VectorSubcoreMesh(core_axis_name="c", subcore_axis_name="s")

    @functools.partial(
        pl.kernel, mesh=mesh,
        out_type=jax.ShapeDtypeStruct((B, sc), jnp.float32),
        scratch_types=[
            pltpu.VMEM((2, CH), jnp.int32),
            pltpu.VMEM((2, CH, sc), jnp.float32),
            pltpu.SemaphoreType.DMA,
            pltpu.SemaphoreType.DMA,
            pltpu.SemaphoreType.DMA,
            pltpu.SemaphoreType.DMA,
        ],
    )
    def _gather(tbl_hbm, idx_hbm, out_hbm, idx_v, rows_v, g0, g1, s0, s1):
        wid = lax.axis_index("s") * 2 + lax.axis_index("c")
        base = wid * bpw
        gs = (g0, g1)
        ss = (s0, s1)
        hg = [None, None]
        hs = [None, None]
        pltpu.sync_copy(idx_hbm.at[pl.ds(base, CH)], idx_v.at[0])
        hg[0] = pltpu.async_copy(tbl_hbm.at[idx_v.at[0]], rows_v.at[0], gs[0])
        for i in range(nch):
            bb = i % 2
            nb = (i + 1) % 2
            if i + 1 < nch:
                if hs[nb] is not None:
                    hs[nb].wait()
                pltpu.sync_copy(idx_hbm.at[pl.ds(base + (i + 1) * CH, CH)],
                                idx_v.at[nb])
                hg[nb] = pltpu.async_copy(tbl_hbm.at[idx_v.at[nb]],
                                          rows_v.at[nb], gs[nb])
            hg[bb].wait()
            hs[bb] = pltpu.async_copy(rows_v.at[bb],
                                      out_hbm.at[pl.ds(base + i * CH, CH)],
                                      ss[bb])
        hs[0].wait()
        hs[1].wait()

    disp = _gather(tbl, idx)

    dispr = disp.reshape(fc, n, l, sc)
    out = pl.pallas_call(
        functools.partial(_proj_body, fc=fc, sc=sc),
        grid=(n, nlb),
        in_specs=[
            pl.BlockSpec((fc, 1, _LB, sc), lambda i, j: (0, i, j, 0)),
            pl.BlockSpec((1, _LB, fc), lambda i, j: (i, j, 0)),
            pl.BlockSpec((h, c), lambda i, j: (0, 0)),
            pl.BlockSpec((1, c), lambda i, j: (0, 0)),
        ],
        out_specs=pl.BlockSpec((1, _LB, c), lambda i, j: (i, j, 0)),
        out_shape=jax.ShapeDtypeStruct((n, l, c), jnp.float32),
    )(dispr, mv, Wm, bmr)
    return out
